# Initial kernel scaffold; baseline (speedup 1.0000x reference)
import jax
import jax.numpy as jnp
from jax.experimental import pallas as pl


def kernel(x, codebook, codebook_avg, cluster_size):
    raise NotImplementedError("write your pallas kernel here")


# speedup vs baseline: 1.1055x; 1.1055x over previous
import functools

import jax
import jax.numpy as jnp
from jax.experimental import pallas as pl
from jax.experimental.pallas import tpu as pltpu

K = 8192
D = 32
N = 16384
DECAY = 0.99
EPS = 1e-05

TN = 512
TK = 4096
NT = N // TN
KT = K // TK


def _argmin_body(xt_ref, cb_ref, a2_ref, b2_ref, out_ref, rmin_ref, rarg_ref):
    kt = pl.program_id(1)

    xt = xt_ref[...]
    cb = cb_ref[...]
    a2 = a2_ref[...]
    b2 = b2_ref[...]
    mm = jnp.dot(xt, cb, preferred_element_type=jnp.float32)
    d = a2 + b2 - 2 * mm
    lmin = jnp.min(d, axis=1, keepdims=True)
    ids = jax.lax.broadcasted_iota(jnp.int32, (TN, TK), 1) + kt * TK
    larg = jnp.min(jnp.where(d == lmin, ids, jnp.int32(K)), axis=1,
                   keepdims=True)

    @pl.when(kt == 0)
    def _first():
        rmin_ref[...] = lmin.astype(jnp.bfloat16).astype(jnp.float32)
        rarg_ref[...] = larg

    @pl.when(kt == KT - 1)
    def _second():
        better = lmin < rmin_ref[...]
        out_ref[...] = jnp.where(better, larg, rarg_ref[...])[:, 0]


@functools.partial(jax.jit, static_argnames=("interpret",))
def _assign(xt, codebook, a2, b2, interpret=False):
    return pl.pallas_call(
        _argmin_body,
        grid=(NT, KT),
        in_specs=[
            pl.BlockSpec((TN, D), lambda n, k: (n, 0)),
            pl.BlockSpec((D, TK), lambda n, k: (0, k)),
            pl.BlockSpec((TN, 1), lambda n, k: (n, 0)),
            pl.BlockSpec((1, TK), lambda n, k: (0, k)),
        ],
        out_specs=pl.BlockSpec((TN,), lambda n, k: (n,)),
        out_shape=jax.ShapeDtypeStruct((N,), jnp.int32),
        scratch_shapes=[
            pltpu.VMEM((TN, 1), jnp.float32),
            pltpu.VMEM((TN, 1), jnp.int32),
        ],
        compiler_params=pltpu.CompilerParams(
            dimension_semantics=("parallel", "arbitrary")),
        interpret=interpret,
    )(xt, codebook, a2, b2)


def kernel(x, codebook, codebook_avg, cluster_size):
    xt = x.T
    a2 = jnp.sum(xt ** 2, axis=-1, keepdims=True)
    b2 = jnp.sum(codebook ** 2, axis=0, keepdims=True)
    idx = _assign(xt.astype(jnp.bfloat16), codebook.astype(jnp.bfloat16),
                  a2, b2)

    z_q = codebook.T[idx]
    z_q = z_q.T

    counts = jnp.bincount(idx, length=K).astype(jnp.float32)
    seg = jax.ops.segment_sum(xt, idx, num_segments=K)

    new_cluster_size = DECAY * cluster_size + (1 - DECAY) * counts
    new_codebook_avg = DECAY * codebook_avg.T + (1 - DECAY) * seg
    n = jnp.sum(new_cluster_size)
    new_cluster_size = (new_cluster_size + EPS) / (n + K * EPS) * n
    new_codebook = codebook_avg.T / new_cluster_size[:, None]
    updates = (new_cluster_size, new_codebook_avg.T, new_codebook.T)
    return (z_q, (updates, idx))


# speedup vs baseline: 1.1939x; 1.0799x over previous
import functools

import jax
import jax.numpy as jnp
from jax import lax
from jax.experimental import pallas as pl
from jax.experimental.pallas import tpu as pltpu
from jax.experimental.pallas import tpu_sc as plsc

K = 8192
D = 32
N = 16384
DECAY = 0.99
EPS = 1e-05

TN = 512
TK = 4096
NT = N // TN
KT = K // TK

AUG = 48

_SC_INFO = plsc.get_sparse_core_info()
NC = _SC_INFO.num_cores
NS = _SC_INFO.num_subcores
L = _SC_INFO.num_lanes
NW = NC * NS
RPW = N // NW
KPS = K // NS


def _argmin_body(xt_ref, cb_ref, a2_ref, b2_ref, out_ref, rmin_ref, rarg_ref):
    kt = pl.program_id(1)

    xt = xt_ref[...]
    cb = cb_ref[...]
    a2 = a2_ref[...]
    b2 = b2_ref[...]
    mm = jnp.dot(xt, cb, preferred_element_type=jnp.float32)
    d = a2 + b2 - 2 * mm
    lmin = jnp.min(d, axis=1, keepdims=True)
    ids = jax.lax.broadcasted_iota(jnp.int32, (TN, TK), 1) + kt * TK
    larg = jnp.min(jnp.where(d == lmin, ids, jnp.int32(K)), axis=1,
                   keepdims=True)

    @pl.when(kt == 0)
    def _first():
        rmin_ref[...] = lmin.astype(jnp.bfloat16).astype(jnp.float32)
        rarg_ref[...] = larg

    @pl.when(kt == KT - 1)
    def _second():
        better = lmin < rmin_ref[...]
        out_ref[...] = jnp.where(better, larg, rarg_ref[...])[:, 0]


@jax.jit
def _assign(xt, codebook, a2, b2):
    return pl.pallas_call(
        _argmin_body,
        grid=(NT, KT),
        in_specs=[
            pl.BlockSpec((TN, D), lambda n, k: (n, 0)),
            pl.BlockSpec((D, TK), lambda n, k: (0, k)),
            pl.BlockSpec((TN, 1), lambda n, k: (n, 0)),
            pl.BlockSpec((1, TK), lambda n, k: (0, k)),
        ],
        out_specs=pl.BlockSpec((TN,), lambda n, k: (n,)),
        out_shape=jax.ShapeDtypeStruct((N,), jnp.int32),
        scratch_shapes=[
            pltpu.VMEM((TN, 1), jnp.float32),
            pltpu.VMEM((TN, 1), jnp.int32),
        ],
        compiler_params=pltpu.CompilerParams(
            dimension_semantics=("parallel", "arbitrary")),
    )(xt, codebook, a2, b2)


def _sc_body(cbt_hbm, xaug_hbm, idx_hbm, zeros_hbm,
             zq_hbm, part_hbm,
             idx_v, zq_v, xaug_v, sh_acc, sem):
    c = lax.axis_index("c")
    s = lax.axis_index("s")
    wid = s * NC + c
    base = wid * RPW

    pltpu.sync_copy(zeros_hbm.at[pl.ds(s * KPS, KPS)],
                    sh_acc.at[pl.ds(s * KPS, KPS)])

    pltpu.sync_copy(idx_hbm.at[pl.ds(base, RPW)], idx_v)

    pltpu.async_copy(cbt_hbm.at[idx_v], zq_v, sem).wait()
    pltpu.sync_copy(zq_v, zq_hbm.at[pl.ds(base, RPW)])

    pltpu.sync_copy(xaug_hbm.at[pl.ds(base, RPW)], xaug_v)

    plsc.subcore_barrier()
    pltpu.sync_copy(xaug_v, sh_acc.at[idx_v], add=True)
    plsc.subcore_barrier()

    pltpu.sync_copy(sh_acc.at[pl.ds(s * KPS, KPS)],
                    part_hbm.at[c, pl.ds(s * KPS, KPS)])


@jax.jit
def _sc_gather_scatter(cbt, xaug, idx, zeros):
    kern = functools.partial(
        pl.kernel,
        mesh=plsc.VectorSubcoreMesh(core_axis_name="c", subcore_axis_name="s"),
        compiler_params=pltpu.CompilerParams(use_tc_tiling_on_sc=False),
        out_type=[
            jax.ShapeDtypeStruct((N, D), jnp.float32),
            jax.ShapeDtypeStruct((NC, K, AUG), jnp.float32),
        ],
        scratch_types=[
            pltpu.VMEM((RPW,), jnp.int32),
            pltpu.VMEM((RPW, D), jnp.float32),
            pltpu.VMEM((RPW, AUG), jnp.float32),
            pltpu.VMEM_SHARED((K, AUG), jnp.float32),
            pltpu.SemaphoreType.DMA,
        ],
    )(_sc_body)
    return kern(cbt, xaug, idx, zeros)


def _ema_body(part_ref, cs_ref, cbavgt_ref, ncs_ref, ncbavgt_ref, ncbt_ref):
    p0 = part_ref[0]
    p1 = part_ref[1]
    seg = p0[:, :D] + p1[:, :D]
    counts = p0[:, D] + p1[:, D]
    cs = cs_ref[...]
    raw = DECAY * cs + (1 - DECAY) * counts
    n = jnp.sum(raw)
    ncs = (raw + EPS) / (n + K * EPS) * n
    ncs_ref[...] = ncs
    cbavgt = cbavgt_ref[...]
    ncbavgt_ref[...] = DECAY * cbavgt + (1 - DECAY) * seg
    ncbt_ref[...] = cbavgt / ncs[:, None]


@jax.jit
def _ema(part, cluster_size, cbavgt):
    return pl.pallas_call(
        _ema_body,
        out_shape=[
            jax.ShapeDtypeStruct((K,), jnp.float32),
            jax.ShapeDtypeStruct((K, D), jnp.float32),
            jax.ShapeDtypeStruct((K, D), jnp.float32),
        ],
    )(part, cluster_size, cbavgt)


def kernel(x, codebook, codebook_avg, cluster_size):
    xt = jnp.permute_dims(x, (1, 0))
    flatten = jnp.reshape(xt, (-1, D))
    a2 = jnp.sum(flatten ** 2, axis=-1, keepdims=True)
    b2 = jnp.sum(codebook ** 2, axis=0, keepdims=True)
    idx = _assign(flatten.astype(jnp.bfloat16), codebook.astype(jnp.bfloat16),
                  a2, b2)

    cbt = codebook.T
    ones_col = jnp.ones((N, 1), jnp.float32)
    pad = jnp.zeros((N, AUG - D - 1), jnp.float32)
    xaug = jnp.concatenate([flatten, ones_col, pad], axis=1)
    zeros = jnp.zeros((K, AUG), jnp.float32)
    zq_rows, part = _sc_gather_scatter(cbt, xaug, idx, zeros)

    ncs, ncbavgt, ncbt = _ema(part, cluster_size, codebook_avg.T)

    z_q = zq_rows.T
    updates = (ncs, ncbavgt.T, ncbt.T)
    return (z_q, (updates, idx))


# speedup vs baseline: 1.2416x; 1.0400x over previous
import functools

import jax
import jax.numpy as jnp
from jax import lax
from jax.experimental import pallas as pl
from jax.experimental.pallas import tpu as pltpu
from jax.experimental.pallas import tpu_sc as plsc

K = 8192
D = 32
N = 16384
DECAY = 0.99
EPS = 1e-05

TN = 1024
TK = 4096
NT = N // TN
KT = K // TK

AUG = 48

_SC_INFO = plsc.get_sparse_core_info()
NC = _SC_INFO.num_cores
NS = _SC_INFO.num_subcores
L = _SC_INFO.num_lanes
NW = NC * NS
RPW = N // NW
KPS = K // NS


def _argmin_body(xt_ref, cb_ref, a2_ref, b2_ref, out_ref, rmin_ref, rarg_ref):
    kt = pl.program_id(1)

    xt = xt_ref[...]
    cb = cb_ref[...]
    a2 = a2_ref[...]
    b2 = b2_ref[...]
    mm = jnp.dot(xt, cb, preferred_element_type=jnp.float32)
    d = a2 + b2 - 2 * mm
    lmin = jnp.min(d, axis=1, keepdims=True)
    ids = jax.lax.broadcasted_iota(jnp.int32, (TN, TK), 1) + kt * TK
    larg = jnp.min(jnp.where(d == lmin, ids, jnp.int32(K)), axis=1,
                   keepdims=True)

    @pl.when(kt == 0)
    def _first():
        rmin_ref[...] = lmin.astype(jnp.bfloat16).astype(jnp.float32)
        rarg_ref[...] = larg

    @pl.when(kt == KT - 1)
    def _second():
        better = lmin < rmin_ref[...]
        out_ref[...] = jnp.where(better, larg, rarg_ref[...])[:, 0]


@jax.jit
def _assign(xt, codebook, a2, b2):
    return pl.pallas_call(
        _argmin_body,
        grid=(NT, KT),
        in_specs=[
            pl.BlockSpec((TN, D), lambda n, k: (n, 0)),
            pl.BlockSpec((D, TK), lambda n, k: (0, k)),
            pl.BlockSpec((TN, 1), lambda n, k: (n, 0)),
            pl.BlockSpec((1, TK), lambda n, k: (0, k)),
        ],
        out_specs=pl.BlockSpec((TN,), lambda n, k: (n,)),
        out_shape=jax.ShapeDtypeStruct((N,), jnp.int32),
        scratch_shapes=[
            pltpu.VMEM((TN, 1), jnp.float32),
            pltpu.VMEM((TN, 1), jnp.int32),
        ],
        compiler_params=pltpu.CompilerParams(
            dimension_semantics=("parallel", "arbitrary")),
    )(xt, codebook, a2, b2)


def _sc_body(cbt_hbm, xaug_hbm, idx_hbm, zeros_hbm,
             zq_hbm, part_hbm,
             idx_v, zq_v, xaug_v, sh_acc, sem):
    c = lax.axis_index("c")
    s = lax.axis_index("s")
    wid = s * NC + c
    base = wid * RPW

    pltpu.sync_copy(zeros_hbm.at[pl.ds(s * KPS, KPS)],
                    sh_acc.at[pl.ds(s * KPS, KPS)])

    pltpu.sync_copy(idx_hbm.at[pl.ds(base, RPW)], idx_v)

    pltpu.async_copy(cbt_hbm.at[idx_v], zq_v, sem).wait()
    pltpu.sync_copy(zq_v, zq_hbm.at[pl.ds(base, RPW)])

    pltpu.sync_copy(xaug_hbm.at[pl.ds(base, RPW)], xaug_v)

    plsc.subcore_barrier()
    pltpu.sync_copy(xaug_v, sh_acc.at[idx_v], add=True)
    plsc.subcore_barrier()

    pltpu.sync_copy(sh_acc.at[pl.ds(s * KPS, KPS)],
                    part_hbm.at[c, pl.ds(s * KPS, KPS)])


@jax.jit
def _sc_gather_scatter(cbt, xaug, idx, zeros):
    kern = functools.partial(
        pl.kernel,
        mesh=plsc.VectorSubcoreMesh(core_axis_name="c", subcore_axis_name="s"),
        compiler_params=pltpu.CompilerParams(use_tc_tiling_on_sc=False),
        out_type=[
            jax.ShapeDtypeStruct((N, D), jnp.float32),
            jax.ShapeDtypeStruct((NC, K, AUG), jnp.float32),
        ],
        scratch_types=[
            pltpu.VMEM((RPW,), jnp.int32),
            pltpu.VMEM((RPW, D), jnp.float32),
            pltpu.VMEM((RPW, AUG), jnp.float32),
            pltpu.VMEM_SHARED((K, AUG), jnp.float32),
            pltpu.SemaphoreType.DMA,
        ],
    )(_sc_body)
    return kern(cbt, xaug, idx, zeros)


def _ema_body(part_ref, cs_ref, cbavgt_ref, ncs_ref, ncbavgt_ref, ncbt_ref):
    p0 = part_ref[0]
    p1 = part_ref[1]
    seg = p0[:, :D] + p1[:, :D]
    counts = p0[:, D] + p1[:, D]
    cs = cs_ref[...]
    raw = DECAY * cs + (1 - DECAY) * counts
    n = jnp.sum(raw)
    ncs = (raw + EPS) / (n + K * EPS) * n
    ncs_ref[...] = ncs
    cbavgt = cbavgt_ref[...]
    ncbavgt_ref[...] = DECAY * cbavgt + (1 - DECAY) * seg
    ncbt_ref[...] = cbavgt / ncs[:, None]


@jax.jit
def _ema(part, cluster_size, cbavgt):
    return pl.pallas_call(
        _ema_body,
        out_shape=[
            jax.ShapeDtypeStruct((K,), jnp.float32),
            jax.ShapeDtypeStruct((K, D), jnp.float32),
            jax.ShapeDtypeStruct((K, D), jnp.float32),
        ],
    )(part, cluster_size, cbavgt)


def kernel(x, codebook, codebook_avg, cluster_size):
    xt = jnp.permute_dims(x, (1, 0))
    flatten = jnp.reshape(xt, (-1, D))
    a2 = jnp.sum(flatten ** 2, axis=-1, keepdims=True)
    b2 = jnp.sum(codebook ** 2, axis=0, keepdims=True)
    idx = _assign(flatten.astype(jnp.bfloat16), codebook.astype(jnp.bfloat16),
                  a2, b2)

    cbt = codebook.T
    ones_col = jnp.ones((N, 1), jnp.float32)
    pad = jnp.zeros((N, AUG - D - 1), jnp.float32)
    xaug = jnp.concatenate([flatten, ones_col, pad], axis=1)
    zeros = jnp.zeros((K, AUG), jnp.float32)
    zq_rows, part = _sc_gather_scatter(cbt, xaug, idx, zeros)

    ncs, ncbavgt, ncbt = _ema(part, cluster_size, codebook_avg.T)

    z_q = zq_rows.T
    updates = (ncs, ncbavgt.T, ncbt.T)
    return (z_q, (updates, idx))
